# baseline (speedup 1.0000x reference)
import functools

import jax
import jax.numpy as jnp
from jax import lax
from jax.experimental import pallas as pl
from jax.experimental.pallas import tpu as pltpu
from jax.experimental.pallas import tpu_sc as plsc

N = 10000
NPAD = 10112
K = 128
NSUB = 16
RPT = NPAD // NSUB
NBUF = 8
LEAD = 5
NCH0 = 80
NCH1 = 80
NCHT = NCH0 + NCH1


def _gather_scatter_pipe(nch, slab0, h_hbm, src_hbm, dst_hbm,
                         acc_sh, sidx, didx, rows, gsems, ssems):
    lag = NBUF - LEAD
    pltpu.sync_copy(src_hbm.at[pl.ds(slab0, nch)], sidx.at[pl.ds(0, nch)])
    pltpu.sync_copy(dst_hbm.at[pl.ds(slab0, nch)], didx.at[pl.ds(0, nch)])
    for b in range(LEAD):
        pltpu.async_copy(h_hbm.at[sidx.at[b]], rows.at[b], gsems[b])

    def turn(i, carry):
        for b in range(NBUF):
            g = i * NBUF + b
            pltpu.make_async_copy(h_hbm.at[sidx.at[g]], rows.at[b],
                                  gsems[b]).wait()
            pltpu.async_copy(rows.at[b], acc_sh.at[didx.at[g]],
                             ssems[b], add=True)
            bn = (b + LEAD) % NBUF

            @pl.when(g >= lag)
            def _():
                pltpu.make_async_copy(rows.at[bn],
                                      acc_sh.at[didx.at[g - lag]],
                                      ssems[bn]).wait()

            @pl.when(g + LEAD < nch)
            def _():
                pltpu.async_copy(h_hbm.at[sidx.at[g + LEAD]], rows.at[bn],
                                 gsems[bn])
        return carry

    lax.fori_loop(0, nch // NBUF, turn, 0)
    for g in range(nch - lag, nch):
        pltpu.make_async_copy(rows.at[g % NBUF], acc_sh.at[didx.at[g]],
                              ssems[g % NBUF]).wait()


def _scatter_body(h_hbm, src_hbm, dst_hbm, zero_hbm, out_hbm,
                  acc_sh, sidx, didx, rows, *sems):
    gsems = sems[:NBUF]
    ssems = sems[NBUF:]
    cid = lax.axis_index("c")
    sid = lax.axis_index("s")

    pltpu.sync_copy(zero_hbm.at[pl.ds(sid * RPT, RPT)],
                    acc_sh.at[pl.ds(sid * RPT, RPT)])
    plsc.subcore_barrier()

    if NCH0 > 0:
        @pl.when(cid == 0)
        def _():
            _gather_scatter_pipe(NCH0, sid * NCH0, h_hbm, src_hbm, dst_hbm,
                                 acc_sh, sidx, didx, rows, gsems, ssems)
    if NCH1 > 0:
        @pl.when(cid == 1)
        def _():
            _gather_scatter_pipe(NCH1, NSUB * NCH0 + sid * NCH1, h_hbm,
                                 src_hbm, dst_hbm, acc_sh, sidx, didx, rows,
                                 gsems, ssems)

    plsc.subcore_barrier()
    row0 = cid * NPAD + sid * RPT
    pltpu.sync_copy(acc_sh.at[pl.ds(sid * RPT, RPT)],
                    out_hbm.at[pl.ds(row0, RPT)])


@functools.lru_cache(maxsize=None)
def _make_scatter(c):
    mesh = plsc.VectorSubcoreMesh(core_axis_name="c", subcore_axis_name="s")
    nchmax = max(NCH0, NCH1)
    return pl.kernel(
        _scatter_body,
        out_type=jax.ShapeDtypeStruct((2 * NPAD, c), jnp.float32),
        mesh=mesh,
        compiler_params=pltpu.CompilerParams(use_tc_tiling_on_sc=False),
        scratch_types=[
            pltpu.VMEM_SHARED((NPAD, c), jnp.float32),
            pltpu.VMEM((nchmax, K), jnp.int32),
            pltpu.VMEM((nchmax, K), jnp.int32),
            pltpu.VMEM((NBUF, K, c), jnp.float32),
        ] + [pltpu.SemaphoreType.DMA] * (2 * NBUF),
    )


def _deg_pipe(nch, slab0, ones_v, dst_hbm, acc_sh, didx, ssem):
    pltpu.sync_copy(dst_hbm.at[pl.ds(slab0, nch)], didx.at[pl.ds(0, nch)])

    def fire(g, carry):
        pltpu.async_copy(ones_v, acc_sh.at[didx.at[g]], ssem, add=True)
        return carry

    lax.fori_loop(0, nch, fire, 0)

    def drain(g, carry):
        pltpu.make_async_copy(ones_v, acc_sh.at[didx.at[g]], ssem).wait()
        return carry

    lax.fori_loop(0, nch, drain, 0)


def _deg_body(ones_hbm, dst_hbm, zero_hbm, out_hbm,
              acc_sh, didx, ones_v, ssem):
    cid = lax.axis_index("c")
    sid = lax.axis_index("s")

    pltpu.sync_copy(ones_hbm, ones_v)
    pltpu.sync_copy(zero_hbm.at[pl.ds(sid * RPT, RPT)],
                    acc_sh.at[pl.ds(sid * RPT, RPT)])
    plsc.subcore_barrier()

    if NCH0 > 0:
        @pl.when(cid == 0)
        def _():
            _deg_pipe(NCH0, sid * NCH0, ones_v, dst_hbm, acc_sh, didx, ssem)
    if NCH1 > 0:
        @pl.when(cid == 1)
        def _():
            _deg_pipe(NCH1, NSUB * NCH0 + sid * NCH1, ones_v, dst_hbm,
                      acc_sh, didx, ssem)

    plsc.subcore_barrier()
    row0 = cid * NPAD + sid * RPT
    pltpu.sync_copy(acc_sh.at[pl.ds(sid * RPT, RPT)],
                    out_hbm.at[pl.ds(row0, RPT)])


@functools.lru_cache(maxsize=None)
def _make_deg():
    mesh = plsc.VectorSubcoreMesh(core_axis_name="c", subcore_axis_name="s")
    nchmax = max(NCH0, NCH1)
    return pl.kernel(
        _deg_body,
        out_type=jax.ShapeDtypeStruct((2 * NPAD, 16), jnp.float32),
        mesh=mesh,
        compiler_params=pltpu.CompilerParams(use_tc_tiling_on_sc=False),
        scratch_types=[
            pltpu.VMEM_SHARED((NPAD, 16), jnp.float32),
            pltpu.VMEM((nchmax, K), jnp.int32),
            pltpu.VMEM((K, 16), jnp.float32),
            pltpu.SemaphoreType.DMA,
        ],
    )


def _dinv(deg_ref):
    deg = deg_ref[:NPAD, 0:1] + deg_ref[NPAD:, 0:1] + 1.0
    row = lax.broadcasted_iota(jnp.int32, (NPAD, 1), 0)
    return jnp.where(row < N, lax.rsqrt(deg), 0.0)


def _tc1_body(deg_ref, x_ref, w1_ref, h1a_ref, h1b_ref):
    h = jnp.dot(x_ref[...], w1_ref[...], preferred_element_type=jnp.float32)
    dinv = _dinv(deg_ref)
    c = h.shape[1] // 2
    h1a_ref[...] = dinv * h[:, :c]
    h1b_ref[...] = dinv * h[:, c:]


def _tc2_body(deg_ref, acc_a_ref, acc_b_ref, h1a_ref, h1b_ref, b1_ref,
              w2_ref, h2p_ref):
    dinv = _dinv(deg_ref)
    c = h1a_ref.shape[1]
    sa = acc_a_ref[:NPAD, :] + acc_a_ref[NPAD:, :] + h1a_ref[...]
    sb = acc_b_ref[:NPAD, :] + acc_b_ref[NPAD:, :] + h1b_ref[...]
    za = jnp.maximum(dinv * sa + b1_ref[...][None, :c], 0.0)
    zb = jnp.maximum(dinv * sb + b1_ref[...][None, c:], 0.0)
    h2 = (jnp.dot(za, w2_ref[:c, :], preferred_element_type=jnp.float32)
          + jnp.dot(zb, w2_ref[c:, :], preferred_element_type=jnp.float32))
    h2p_ref[...] = dinv * h2


def _tc3_body(deg_ref, acc_ref, h2p_ref, b2_ref, out_ref):
    dinv = _dinv(deg_ref)
    s = dinv * (acc_ref[:NPAD, :] + acc_ref[NPAD:, :] + h2p_ref[...])
    s = s + b2_ref[...][None, :]
    m = jnp.max(s, axis=1, keepdims=True)
    sh = s - m
    out_ref[...] = sh - jnp.log(jnp.sum(jnp.exp(sh), axis=1, keepdims=True))


def kernel(x, edge_index, W1, b1, W2, b2):
    e = edge_index.shape[1]
    epad = NSUB * NCHT * K
    assert epad >= e

    src = edge_index[0].astype(jnp.int32)
    dst = edge_index[1].astype(jnp.int32)
    pad = jnp.full((epad - e,), N, jnp.int32)
    srcp = jnp.concatenate([src, pad]).reshape(NSUB * NCHT, K)
    dstp = jnp.concatenate([dst, pad]).reshape(NSUB * NCHT, K)
    xpad = jnp.pad(x, ((0, NPAD - N), (0, 0)))

    hid = W1.shape[1]
    half = hid // 2
    out_ch = W2.shape[1]
    ones_k = jnp.ones((K, 16), jnp.float32)
    z16 = jnp.zeros((NPAD, 16), jnp.float32)
    zhalf = jnp.zeros((NPAD, half), jnp.float32)
    zo = jnp.zeros((NPAD, out_ch), jnp.float32)

    deg16 = _make_deg()(ones_k, dstp, z16)

    h1a, h1b = pl.pallas_call(
        _tc1_body,
        out_shape=[jax.ShapeDtypeStruct((NPAD, half), jnp.float32),
                   jax.ShapeDtypeStruct((NPAD, half), jnp.float32)],
    )(deg16, xpad, W1)

    acc1a = _make_scatter(half)(h1a, srcp, dstp, zhalf)
    acc1b = _make_scatter(half)(h1b, srcp, dstp, zhalf)

    h2p = pl.pallas_call(
        _tc2_body,
        out_shape=jax.ShapeDtypeStruct((NPAD, out_ch), jnp.float32),
    )(deg16, acc1a, acc1b, h1a, h1b, b1, W2)

    acc2 = _make_scatter(out_ch)(h2p, srcp, dstp, zo)

    outp = pl.pallas_call(
        _tc3_body,
        out_shape=jax.ShapeDtypeStruct((NPAD, out_ch), jnp.float32),
    )(deg16, acc2, h2p, b2)

    return outp[:N]


# speedup vs baseline: 2.4130x; 2.4130x over previous
import functools

import jax
import jax.numpy as jnp
from jax import lax
from jax.experimental import pallas as pl
from jax.experimental.pallas import tpu as pltpu
from jax.experimental.pallas import tpu_sc as plsc

N = 10000
NPAD = 10112
K = 128
NSUB = 16
RPT = NPAD // NSUB
NBUF = 5
LEAD = 3
NCH0 = 120
NCH1 = 40
NCHT = NCH0 + NCH1


def _gather_scatter_pipe(nch, slab0, h_hbm, src_hbm, dst_hbm,
                         acc_sh, sidx, didx, rows, gsems, ssems):
    lag = NBUF - LEAD
    pltpu.sync_copy(src_hbm.at[pl.ds(slab0, nch)], sidx.at[pl.ds(0, nch)])
    pltpu.sync_copy(dst_hbm.at[pl.ds(slab0, nch)], didx.at[pl.ds(0, nch)])
    for b in range(LEAD):
        pltpu.async_copy(h_hbm.at[sidx.at[b]], rows.at[b], gsems[b])

    def turn(i, carry):
        for b in range(NBUF):
            g = i * NBUF + b
            pltpu.make_async_copy(h_hbm.at[sidx.at[g]], rows.at[b],
                                  gsems[b]).wait()
            pltpu.async_copy(rows.at[b], acc_sh.at[didx.at[g]],
                             ssems[b], add=True)
            bn = (b + LEAD) % NBUF

            @pl.when(g >= lag)
            def _():
                pltpu.make_async_copy(rows.at[bn],
                                      acc_sh.at[didx.at[g - lag]],
                                      ssems[bn]).wait()

            @pl.when(g + LEAD < nch)
            def _():
                pltpu.async_copy(h_hbm.at[sidx.at[g + LEAD]], rows.at[bn],
                                 gsems[bn])
        return carry

    lax.fori_loop(0, nch // NBUF, turn, 0)
    for g in range(nch - lag, nch):
        pltpu.make_async_copy(rows.at[g % NBUF], acc_sh.at[didx.at[g]],
                              ssems[g % NBUF]).wait()


def _scatter_body(h_hbm, src_hbm, dst_hbm, zero_hbm, out_hbm,
                  acc_sh, sidx, didx, rows, *sems):
    gsems = sems[:NBUF]
    ssems = sems[NBUF:]
    cid = lax.axis_index("c")
    sid = lax.axis_index("s")

    pltpu.sync_copy(zero_hbm.at[pl.ds(sid * RPT, RPT)],
                    acc_sh.at[pl.ds(sid * RPT, RPT)])
    plsc.subcore_barrier()

    if NCH0 > 0:
        @pl.when(cid == 0)
        def _():
            _gather_scatter_pipe(NCH0, sid * NCH0, h_hbm, src_hbm, dst_hbm,
                                 acc_sh, sidx, didx, rows, gsems, ssems)
    if NCH1 > 0:
        @pl.when(cid == 1)
        def _():
            _gather_scatter_pipe(NCH1, NSUB * NCH0 + sid * NCH1, h_hbm,
                                 src_hbm, dst_hbm, acc_sh, sidx, didx, rows,
                                 gsems, ssems)

    plsc.subcore_barrier()
    row0 = cid * NPAD + sid * RPT
    pltpu.sync_copy(acc_sh.at[pl.ds(sid * RPT, RPT)],
                    out_hbm.at[pl.ds(row0, RPT)])


@functools.lru_cache(maxsize=None)
def _make_scatter(c):
    mesh = plsc.VectorSubcoreMesh(core_axis_name="c", subcore_axis_name="s")
    nchmax = max(NCH0, NCH1)
    return pl.kernel(
        _scatter_body,
        out_type=jax.ShapeDtypeStruct((2 * NPAD, c), jnp.float32),
        mesh=mesh,
        compiler_params=pltpu.CompilerParams(use_tc_tiling_on_sc=False),
        scratch_types=[
            pltpu.VMEM_SHARED((NPAD, c), jnp.float32),
            pltpu.VMEM((nchmax, K), jnp.int32),
            pltpu.VMEM((nchmax, K), jnp.int32),
            pltpu.VMEM((NBUF, K, c), jnp.float32),
        ] + [pltpu.SemaphoreType.DMA] * (2 * NBUF),
    )


def _deg_pipe(nch, slab0, ones_v, dst_hbm, acc_sh, didx, ssem):
    pltpu.sync_copy(dst_hbm.at[pl.ds(slab0, nch)], didx.at[pl.ds(0, nch)])

    def fire(g, carry):
        pltpu.async_copy(ones_v, acc_sh.at[didx.at[g]], ssem, add=True)
        return carry

    lax.fori_loop(0, nch, fire, 0)

    def drain(g, carry):
        pltpu.make_async_copy(ones_v, acc_sh.at[didx.at[g]], ssem).wait()
        return carry

    lax.fori_loop(0, nch, drain, 0)


def _deg_body(ones_hbm, dst_hbm, zero_hbm, out_hbm,
              acc_sh, didx, ones_v, ssem):
    cid = lax.axis_index("c")
    sid = lax.axis_index("s")

    pltpu.sync_copy(ones_hbm, ones_v)
    pltpu.sync_copy(zero_hbm.at[pl.ds(sid * RPT, RPT)],
                    acc_sh.at[pl.ds(sid * RPT, RPT)])
    plsc.subcore_barrier()

    if NCH0 > 0:
        @pl.when(cid == 0)
        def _():
            _deg_pipe(NCH0, sid * NCH0, ones_v, dst_hbm, acc_sh, didx, ssem)
    if NCH1 > 0:
        @pl.when(cid == 1)
        def _():
            _deg_pipe(NCH1, NSUB * NCH0 + sid * NCH1, ones_v, dst_hbm,
                      acc_sh, didx, ssem)

    plsc.subcore_barrier()
    row0 = cid * NPAD + sid * RPT
    pltpu.sync_copy(acc_sh.at[pl.ds(sid * RPT, RPT)],
                    out_hbm.at[pl.ds(row0, RPT)])


@functools.lru_cache(maxsize=None)
def _make_deg():
    mesh = plsc.VectorSubcoreMesh(core_axis_name="c", subcore_axis_name="s")
    nchmax = max(NCH0, NCH1)
    return pl.kernel(
        _deg_body,
        out_type=jax.ShapeDtypeStruct((2 * NPAD, 16), jnp.float32),
        mesh=mesh,
        compiler_params=pltpu.CompilerParams(use_tc_tiling_on_sc=False),
        scratch_types=[
            pltpu.VMEM_SHARED((NPAD, 16), jnp.float32),
            pltpu.VMEM((nchmax, K), jnp.int32),
            pltpu.VMEM((K, 16), jnp.float32),
            pltpu.SemaphoreType.DMA,
        ],
    )


def _dinv(deg_ref):
    deg = deg_ref[:NPAD, 0:1] + deg_ref[NPAD:, 0:1] + 1.0
    row = lax.broadcasted_iota(jnp.int32, (NPAD, 1), 0)
    return jnp.where(row < N, lax.rsqrt(deg), 0.0)


def _tc1_body(deg_ref, x_ref, w1_ref, h1a_ref, h1b_ref):
    h = jnp.dot(x_ref[...], w1_ref[...], preferred_element_type=jnp.float32)
    dinv = _dinv(deg_ref)
    c = h.shape[1] // 2
    h1a_ref[...] = dinv * h[:, :c]
    h1b_ref[...] = dinv * h[:, c:]


def _tc2_body(deg_ref, acc_a_ref, acc_b_ref, h1a_ref, h1b_ref, b1_ref,
              w2_ref, h2p_ref):
    dinv = _dinv(deg_ref)
    c = h1a_ref.shape[1]
    sa = acc_a_ref[:NPAD, :] + acc_a_ref[NPAD:, :] + h1a_ref[...]
    sb = acc_b_ref[:NPAD, :] + acc_b_ref[NPAD:, :] + h1b_ref[...]
    za = jnp.maximum(dinv * sa + b1_ref[...][None, :c], 0.0)
    zb = jnp.maximum(dinv * sb + b1_ref[...][None, c:], 0.0)
    h2 = (jnp.dot(za, w2_ref[:c, :], preferred_element_type=jnp.float32)
          + jnp.dot(zb, w2_ref[c:, :], preferred_element_type=jnp.float32))
    h2p_ref[...] = dinv * h2


def _tc3_body(deg_ref, acc_ref, h2p_ref, b2_ref, out_ref):
    dinv = _dinv(deg_ref)
    s = dinv * (acc_ref[:NPAD, :] + acc_ref[NPAD:, :] + h2p_ref[...])
    s = s + b2_ref[...][None, :]
    m = jnp.max(s, axis=1, keepdims=True)
    sh = s - m
    out_ref[...] = sh - jnp.log(jnp.sum(jnp.exp(sh), axis=1, keepdims=True))


def kernel(x, edge_index, W1, b1, W2, b2):
    e = edge_index.shape[1]
    epad = NSUB * NCHT * K
    assert epad >= e

    src = edge_index[0].astype(jnp.int32)
    dst = edge_index[1].astype(jnp.int32)
    pad = N + jnp.arange(epad - e, dtype=jnp.int32) % (NPAD - N)
    srcp = jnp.concatenate([src, pad]).reshape(NSUB * NCHT, K)
    dstp = jnp.concatenate([dst, pad]).reshape(NSUB * NCHT, K)
    xpad = jnp.pad(x, ((0, NPAD - N), (0, 0)))

    hid = W1.shape[1]
    half = hid // 2
    out_ch = W2.shape[1]
    ones_k = jnp.ones((K, 16), jnp.float32)
    z16 = jnp.zeros((NPAD, 16), jnp.float32)
    zhalf = jnp.zeros((NPAD, half), jnp.float32)
    zo = jnp.zeros((NPAD, out_ch), jnp.float32)

    deg16 = _make_deg()(ones_k, dstp, z16)

    h1a, h1b = pl.pallas_call(
        _tc1_body,
        out_shape=[jax.ShapeDtypeStruct((NPAD, half), jnp.float32),
                   jax.ShapeDtypeStruct((NPAD, half), jnp.float32)],
    )(deg16, xpad, W1)

    acc1a = _make_scatter(half)(h1a, srcp, dstp, zhalf)
    acc1b = _make_scatter(half)(h1b, srcp, dstp, zhalf)

    h2p = pl.pallas_call(
        _tc2_body,
        out_shape=jax.ShapeDtypeStruct((NPAD, out_ch), jnp.float32),
    )(deg16, acc1a, acc1b, h1a, h1b, b1, W2)

    acc2 = _make_scatter(out_ch)(h2p, srcp, dstp, zo)

    outp = pl.pallas_call(
        _tc3_body,
        out_shape=jax.ShapeDtypeStruct((NPAD, out_ch), jnp.float32),
    )(deg16, acc2, h2p, b2)

    return outp[:N]


# speedup vs baseline: 2.9192x; 1.2098x over previous
import functools

import jax
import jax.numpy as jnp
from jax import lax
from jax.experimental import pallas as pl
from jax.experimental.pallas import tpu as pltpu
from jax.experimental.pallas import tpu_sc as plsc

N = 10000
NPAD = 10112
K = 128
NSUB = 16
RPT = NPAD // NSUB
NBUF = 5
LEAD = 3
NCH0 = 80
NCH1 = 80
NCHT = NCH0 + NCH1


def _gather_scatter_pipe(nch, slab0, h_hbm, src_hbm, dst_hbm,
                         acc_sh, sidx, didx, rows, gsems, ssems):
    lag = NBUF - LEAD
    pltpu.sync_copy(src_hbm.at[pl.ds(slab0, nch)], sidx.at[pl.ds(0, nch)])
    pltpu.sync_copy(dst_hbm.at[pl.ds(slab0, nch)], didx.at[pl.ds(0, nch)])
    for b in range(LEAD):
        pltpu.async_copy(h_hbm.at[sidx.at[b]], rows.at[b], gsems[b])

    def turn(i, carry):
        for b in range(NBUF):
            g = i * NBUF + b
            pltpu.make_async_copy(h_hbm.at[sidx.at[g]], rows.at[b],
                                  gsems[b]).wait()
            pltpu.async_copy(rows.at[b], acc_sh.at[didx.at[g]],
                             ssems[b], add=True)
            bn = (b + LEAD) % NBUF

            @pl.when(g >= lag)
            def _():
                pltpu.make_async_copy(rows.at[bn],
                                      acc_sh.at[didx.at[g - lag]],
                                      ssems[bn]).wait()

            @pl.when(g + LEAD < nch)
            def _():
                pltpu.async_copy(h_hbm.at[sidx.at[g + LEAD]], rows.at[bn],
                                 gsems[bn])
        return carry

    lax.fori_loop(0, nch // NBUF, turn, 0)
    for g in range(nch - lag, nch):
        pltpu.make_async_copy(rows.at[g % NBUF], acc_sh.at[didx.at[g]],
                              ssems[g % NBUF]).wait()


def _scatter_body(h_hbm, src_hbm, dst_hbm, zero_hbm, out_hbm,
                  acc_sh, sidx, didx, rows, *sems):
    gsems = sems[:NBUF]
    ssems = sems[NBUF:]
    cid = lax.axis_index("c")
    sid = lax.axis_index("s")

    pltpu.sync_copy(zero_hbm.at[pl.ds(sid * RPT, RPT)],
                    acc_sh.at[pl.ds(sid * RPT, RPT)])
    plsc.subcore_barrier()

    if NCH0 > 0:
        @pl.when(cid == 0)
        def _():
            _gather_scatter_pipe(NCH0, sid * NCH0, h_hbm, src_hbm, dst_hbm,
                                 acc_sh, sidx, didx, rows, gsems, ssems)
    if NCH1 > 0:
        @pl.when(cid == 1)
        def _():
            _gather_scatter_pipe(NCH1, NSUB * NCH0 + sid * NCH1, h_hbm,
                                 src_hbm, dst_hbm, acc_sh, sidx, didx, rows,
                                 gsems, ssems)

    plsc.subcore_barrier()
    row0 = cid * NPAD + sid * RPT
    pltpu.sync_copy(acc_sh.at[pl.ds(sid * RPT, RPT)],
                    out_hbm.at[pl.ds(row0, RPT)])


@functools.lru_cache(maxsize=None)
def _make_scatter(c):
    mesh = plsc.VectorSubcoreMesh(core_axis_name="c", subcore_axis_name="s")
    nchmax = max(NCH0, NCH1)
    return pl.kernel(
        _scatter_body,
        out_type=jax.ShapeDtypeStruct((2 * NPAD, c), jnp.float32),
        mesh=mesh,
        compiler_params=pltpu.CompilerParams(use_tc_tiling_on_sc=False),
        scratch_types=[
            pltpu.VMEM_SHARED((NPAD, c), jnp.float32),
            pltpu.VMEM((nchmax, K), jnp.int32),
            pltpu.VMEM((nchmax, K), jnp.int32),
            pltpu.VMEM((NBUF, K, c), jnp.float32),
        ] + [pltpu.SemaphoreType.DMA] * (2 * NBUF),
    )


def _deg_pipe(nch, slab0, ones_v, dst_hbm, acc_sh, didx, ssem):
    pltpu.sync_copy(dst_hbm.at[pl.ds(slab0, nch)], didx.at[pl.ds(0, nch)])

    def fire(g, carry):
        pltpu.async_copy(ones_v, acc_sh.at[didx.at[g]], ssem, add=True)
        return carry

    lax.fori_loop(0, nch, fire, 0)

    def drain(g, carry):
        pltpu.make_async_copy(ones_v, acc_sh.at[didx.at[g]], ssem).wait()
        return carry

    lax.fori_loop(0, nch, drain, 0)


def _deg_body(ones_hbm, dst_hbm, zero_hbm, out_hbm,
              acc_sh, didx, ones_v, ssem):
    cid = lax.axis_index("c")
    sid = lax.axis_index("s")

    pltpu.sync_copy(ones_hbm, ones_v)
    pltpu.sync_copy(zero_hbm.at[pl.ds(sid * RPT, RPT)],
                    acc_sh.at[pl.ds(sid * RPT, RPT)])
    plsc.subcore_barrier()

    if NCH0 > 0:
        @pl.when(cid == 0)
        def _():
            _deg_pipe(NCH0, sid * NCH0, ones_v, dst_hbm, acc_sh, didx, ssem)
    if NCH1 > 0:
        @pl.when(cid == 1)
        def _():
            _deg_pipe(NCH1, NSUB * NCH0 + sid * NCH1, ones_v, dst_hbm,
                      acc_sh, didx, ssem)

    plsc.subcore_barrier()
    row0 = cid * NPAD + sid * RPT
    pltpu.sync_copy(acc_sh.at[pl.ds(sid * RPT, RPT)],
                    out_hbm.at[pl.ds(row0, RPT)])


@functools.lru_cache(maxsize=None)
def _make_deg():
    mesh = plsc.VectorSubcoreMesh(core_axis_name="c", subcore_axis_name="s")
    nchmax = max(NCH0, NCH1)
    return pl.kernel(
        _deg_body,
        out_type=jax.ShapeDtypeStruct((2 * NPAD, 16), jnp.float32),
        mesh=mesh,
        compiler_params=pltpu.CompilerParams(use_tc_tiling_on_sc=False),
        scratch_types=[
            pltpu.VMEM_SHARED((NPAD, 16), jnp.float32),
            pltpu.VMEM((nchmax, K), jnp.int32),
            pltpu.VMEM((K, 16), jnp.float32),
            pltpu.SemaphoreType.DMA,
        ],
    )


def _dinv(deg_ref):
    deg = deg_ref[:NPAD, 0:1] + deg_ref[NPAD:, 0:1] + 1.0
    row = lax.broadcasted_iota(jnp.int32, (NPAD, 1), 0)
    return jnp.where(row < N, lax.rsqrt(deg), 0.0)


def _tc1_body(deg_ref, x_ref, w1_ref, h1a_ref, h1b_ref):
    h = jnp.dot(x_ref[...], w1_ref[...], preferred_element_type=jnp.float32)
    dinv = _dinv(deg_ref)
    c = h.shape[1] // 2
    h1a_ref[...] = dinv * h[:, :c]
    h1b_ref[...] = dinv * h[:, c:]


def _tc2_body(deg_ref, acc_a_ref, acc_b_ref, h1a_ref, h1b_ref, b1_ref,
              w2_ref, h2p_ref):
    dinv = _dinv(deg_ref)
    c = h1a_ref.shape[1]
    sa = acc_a_ref[:NPAD, :] + acc_a_ref[NPAD:, :] + h1a_ref[...]
    sb = acc_b_ref[:NPAD, :] + acc_b_ref[NPAD:, :] + h1b_ref[...]
    za = jnp.maximum(dinv * sa + b1_ref[...][None, :c], 0.0)
    zb = jnp.maximum(dinv * sb + b1_ref[...][None, c:], 0.0)
    h2 = (jnp.dot(za, w2_ref[:c, :], preferred_element_type=jnp.float32)
          + jnp.dot(zb, w2_ref[c:, :], preferred_element_type=jnp.float32))
    h2p_ref[...] = dinv * h2


def _tc3_body(deg_ref, acc_ref, h2p_ref, b2_ref, out_ref):
    dinv = _dinv(deg_ref)
    s = dinv * (acc_ref[:NPAD, :] + acc_ref[NPAD:, :] + h2p_ref[...])
    s = s + b2_ref[...][None, :]
    m = jnp.max(s, axis=1, keepdims=True)
    sh = s - m
    out_ref[...] = sh - jnp.log(jnp.sum(jnp.exp(sh), axis=1, keepdims=True))


def kernel(x, edge_index, W1, b1, W2, b2):
    e = edge_index.shape[1]
    epad = NSUB * NCHT * K
    assert epad >= e

    src = edge_index[0].astype(jnp.int32)
    dst = edge_index[1].astype(jnp.int32)
    pad = N + jnp.arange(epad - e, dtype=jnp.int32) % (NPAD - N)
    srcp = jnp.concatenate([src, pad]).reshape(NSUB * NCHT, K)
    dstp = jnp.concatenate([dst, pad]).reshape(NSUB * NCHT, K)
    xpad = jnp.pad(x, ((0, NPAD - N), (0, 0)))

    hid = W1.shape[1]
    half = hid // 2
    out_ch = W2.shape[1]
    ones_k = jnp.ones((K, 16), jnp.float32)
    z16 = jnp.zeros((NPAD, 16), jnp.float32)
    zhalf = jnp.zeros((NPAD, half), jnp.float32)
    zo = jnp.zeros((NPAD, out_ch), jnp.float32)

    deg16 = _make_deg()(ones_k, dstp, z16)

    h1a, h1b = pl.pallas_call(
        _tc1_body,
        out_shape=[jax.ShapeDtypeStruct((NPAD, half), jnp.float32),
                   jax.ShapeDtypeStruct((NPAD, half), jnp.float32)],
    )(deg16, xpad, W1)

    acc1a = _make_scatter(half)(h1a, srcp, dstp, zhalf)
    acc1b = _make_scatter(half)(h1b, srcp, dstp, zhalf)

    h2p = pl.pallas_call(
        _tc2_body,
        out_shape=jax.ShapeDtypeStruct((NPAD, out_ch), jnp.float32),
    )(deg16, acc1a, acc1b, h1a, h1b, b1, W2)

    acc2 = _make_scatter(out_ch)(h2p, srcp, dstp, zo)

    outp = pl.pallas_call(
        _tc3_body,
        out_shape=jax.ShapeDtypeStruct((NPAD, out_ch), jnp.float32),
    )(deg16, acc2, h2p, b2)

    return outp[:N]
